# Initial kernel scaffold; baseline (speedup 1.0000x reference)
import jax
import jax.numpy as jnp
from jax.experimental import pallas as pl


def kernel(action, W1, b1, W2, b2, W3, b3, E, W4, b4, W5, b5, W6, b6):
    raise NotImplementedError("write your pallas kernel here")


# speedup vs baseline: 1.4352x; 1.4352x over previous
import functools

import jax
import jax.numpy as jnp
from jax import lax
from jax.experimental import pallas as pl
from jax.experimental.pallas import tpu as pltpu
from jax.experimental.pallas import tpu_sc as plsc

_B = 16384
_A = 6
_H = 256
_D = 256
_K = 8192

_BB = 2048
_NB = _B // _BB
_CK = 1024
_NC = _K // _CK

_NW = 32
_BPW = _B // _NW
_GCH = 128
_NGC = _BPW // _GCH


def _enc_vq_body(act_ref, w1_ref, b1_ref, w2_ref, b2_ref, w3_ref, b3_ref,
                 ebf_ref, enc_ref, idx_ref):
    f32 = jnp.float32
    cdims = (((1,), (1,)), ((), ()))
    x = act_ref[...]
    x = jnp.maximum(
        lax.dot_general(x, w1_ref[...], cdims, preferred_element_type=f32)
        + b1_ref[...], 0.0)
    x = jnp.maximum(
        lax.dot_general(x, w2_ref[...], cdims, preferred_element_type=f32)
        + b2_ref[...], 0.0)
    enc = (lax.dot_general(x, w3_ref[...], cdims, preferred_element_type=f32)
           + b3_ref[...])
    enc_ref[...] = enc

    enc_bf = enc.astype(jnp.bfloat16)
    ones_row = jnp.ones((1, _D), dtype=f32)
    best = jnp.full((_BB, 1), jnp.inf, dtype=f32)
    best_i = jnp.zeros((_BB, 1), dtype=jnp.int32)
    for c in range(_NC):
        ec = ebf_ref[c * _CK:(c + 1) * _CK, :]
        ecf = ec.astype(f32)
        n2 = lax.dot_general(ones_row, ecf * ecf, cdims,
                             preferred_element_type=f32)
        dots = lax.dot_general(enc_bf, ec, cdims,
                               preferred_element_type=f32)
        s = n2 - 2.0 * dots
        cm = jnp.min(s, axis=1, keepdims=True)
        io = lax.broadcasted_iota(jnp.int32, (_BB, _CK), 1)
        ci = (jnp.min(jnp.where(s == cm, io, _CK), axis=1, keepdims=True)
              + c * _CK)
        upd = cm < best
        best = jnp.where(upd, cm, best)
        best_i = jnp.where(upd, ci, best_i)
    idx_ref[0] = best_i


def _dec_loss_body(q_ref, enc_ref, act_ref, w4_ref, b4_ref, w5_ref, b5_ref,
                   w6_ref, b6_ref, rs_ref, qs_ref):
    f32 = jnp.float32
    cdims = (((1,), (1,)), ((), ()))
    i = pl.program_id(0)
    q = q_ref[...]
    h = jnp.maximum(
        lax.dot_general(q, w4_ref[...], cdims, preferred_element_type=f32)
        + b4_ref[...], 0.0)
    h = jnp.maximum(
        lax.dot_general(h, w5_ref[...], cdims, preferred_element_type=f32)
        + b5_ref[...], 0.0)
    r = jnp.tanh(
        lax.dot_general(h, w6_ref[...], cdims, preferred_element_type=f32)
        + b6_ref[...])
    dr = r - act_ref[...]
    dq = enc_ref[...] - q
    pr = jnp.sum(jnp.sum(dr * dr, axis=1, keepdims=True), axis=0,
                 keepdims=True)
    pq = jnp.sum(jnp.sum(dq * dq, axis=1, keepdims=True), axis=0,
                 keepdims=True)

    @pl.when(i == 0)
    def _():
        rs_ref[...] = pr
        qs_ref[...] = pq

    @pl.when(i != 0)
    def _():
        rs_ref[...] += pr
        qs_ref[...] += pq


def _sc_gather(e, idx):
    mesh = plsc.VectorSubcoreMesh(core_axis_name="c", subcore_axis_name="s")

    @functools.partial(
        pl.kernel, mesh=mesh,
        out_type=jax.ShapeDtypeStruct((_B, _D), jnp.float32),
        scratch_types=[
            pltpu.VMEM((_NGC, _GCH), jnp.int32),
            pltpu.VMEM((_GCH, _D), jnp.float32),
            pltpu.SemaphoreType.DMA,
        ],
    )
    def gk(e_hbm, idx_hbm, out_hbm, idx_v, rows_v, sem):
        wid = lax.axis_index("s") * 2 + lax.axis_index("c")
        base = wid * _BPW
        for c in range(_NGC):
            pltpu.sync_copy(idx_hbm.at[pl.ds(base + c * _GCH, _GCH)],
                            idx_v.at[c])
            pltpu.async_copy(e_hbm.at[idx_v.at[c]], rows_v, sem).wait()
            pltpu.sync_copy(rows_v,
                            out_hbm.at[pl.ds(base + c * _GCH, _GCH)])

    return gk(e, idx)


def kernel(action, W1, b1, W2, b2, W3, b3, E, W4, b4, W5, b5, W6, b6):
    b1r = b1.reshape(1, _H)
    b2r = b2.reshape(1, _H)
    b3r = b3.reshape(1, _D)
    b4r = b4.reshape(1, _H)
    b5r = b5.reshape(1, _H)
    b6r = b6.reshape(1, _A)
    ebf = E.astype(jnp.bfloat16)

    enc, idx3 = pl.pallas_call(
        _enc_vq_body,
        grid=(_NB,),
        in_specs=[
            pl.BlockSpec((_BB, _A), lambda i: (i, 0)),
            pl.BlockSpec((_H, _A), lambda i: (0, 0)),
            pl.BlockSpec((1, _H), lambda i: (0, 0)),
            pl.BlockSpec((_H, _H), lambda i: (0, 0)),
            pl.BlockSpec((1, _H), lambda i: (0, 0)),
            pl.BlockSpec((_D, _H), lambda i: (0, 0)),
            pl.BlockSpec((1, _D), lambda i: (0, 0)),
            pl.BlockSpec((_K, _D), lambda i: (0, 0)),
        ],
        out_specs=[
            pl.BlockSpec((_BB, _D), lambda i: (i, 0)),
            pl.BlockSpec((1, _BB, 1), lambda i: (i, 0, 0)),
        ],
        out_shape=[
            jax.ShapeDtypeStruct((_B, _D), jnp.float32),
            jax.ShapeDtypeStruct((_NB, _BB, 1), jnp.int32),
        ],
    )(action, W1, b1r, W2, b2r, W3, b3r, ebf)

    idx = idx3.reshape(_B)
    q = _sc_gather(E, idx)

    rs, qs = pl.pallas_call(
        _dec_loss_body,
        grid=(_NB,),
        in_specs=[
            pl.BlockSpec((_BB, _D), lambda i: (i, 0)),
            pl.BlockSpec((_BB, _D), lambda i: (i, 0)),
            pl.BlockSpec((_BB, _A), lambda i: (i, 0)),
            pl.BlockSpec((_H, _D), lambda i: (0, 0)),
            pl.BlockSpec((1, _H), lambda i: (0, 0)),
            pl.BlockSpec((_H, _H), lambda i: (0, 0)),
            pl.BlockSpec((1, _H), lambda i: (0, 0)),
            pl.BlockSpec((_A, _H), lambda i: (0, 0)),
            pl.BlockSpec((1, _A), lambda i: (0, 0)),
        ],
        out_specs=[
            pl.BlockSpec((1, 1), lambda i: (0, 0)),
            pl.BlockSpec((1, 1), lambda i: (0, 0)),
        ],
        out_shape=[
            jax.ShapeDtypeStruct((1, 1), jnp.float32),
            jax.ShapeDtypeStruct((1, 1), jnp.float32),
        ],
    )(q, enc, action, W4, b4r, W5, b5r, W6, b6r)

    return rs[0, 0] / (_B * _A) + 1.25 * qs[0, 0] / (_B * _D)
